# Initial kernel scaffold; baseline (speedup 1.0000x reference)
import jax
import jax.numpy as jnp
from jax.experimental import pallas as pl


def kernel(x, edge_index, edge_weight, W, b, temp):
    raise NotImplementedError("write your pallas kernel here")


# speedup vs baseline: 2.8137x; 2.8137x over previous
import functools

import jax
import jax.numpy as jnp
from jax import lax
from jax.experimental import pallas as pl
from jax.experimental.pallas import tpu as pltpu
from jax.experimental.pallas import tpu_sc as plsc

N = 10000
E = 320000
D = 128
L = 10

NC = 1
NS = 16
NW = NC * NS
EPW = E // NW
C = 80
SB = 10
NSC = EPW // (SB * C)
RPT = 624
REM = N - NS * RPT


def _sc_aggregate(hl, src_r, dst_r, w_r):
    mesh = plsc.VectorSubcoreMesh(core_axis_name="c", subcore_axis_name="s",
                                  num_cores=NC)

    @functools.partial(
        pl.kernel,
        mesh=mesh,
        out_type=jax.ShapeDtypeStruct((NC, N, D), jnp.float32),
        scratch_types=[
            pltpu.VMEM((SB, C), jnp.int32),
            pltpu.VMEM((SB, C), jnp.int32),
            pltpu.VMEM((SB, C), jnp.float32),
            pltpu.VMEM((C, D), jnp.float32),
            pltpu.VMEM_SHARED((N, D), jnp.float32),
            pltpu.SemaphoreType.DMA,
        ],
    )
    def k(hl_hbm, src_hbm, dst_hbm, w_hbm, out_hbm,
          src_v, dst_v, w_v, rows_v, acc, gsem):
        c = lax.axis_index("c")
        s = lax.axis_index("s")
        wid = c * NS + s

        def zrow_body(r, carry):
            for kk in range(D // 16):
                rows_v[r, pl.ds(kk * 16, 16)] = jnp.zeros((16,), jnp.float32)
            return carry
        lax.fori_loop(0, C, zrow_body, 0)
        for t in range(RPT // C):
            pltpu.sync_copy(rows_v, acc.at[pl.ds(s * RPT + t * C, C)])
        pltpu.sync_copy(rows_v.at[pl.ds(0, RPT % C)],
                        acc.at[pl.ds(s * RPT + (RPT // C) * C, RPT % C)])

        @pl.when(s == 0)
        def _zero_rem():
            pltpu.sync_copy(rows_v.at[pl.ds(0, REM)],
                            acc.at[pl.ds(NS * RPT, REM)])
        plsc.subcore_barrier()

        def super_body(t, carry):
            pltpu.sync_copy(src_hbm.at[wid, t], src_v)
            pltpu.sync_copy(dst_hbm.at[wid, t], dst_v)
            pltpu.sync_copy(w_hbm.at[wid, t], w_v)

            def chunk_body(j, carry2):
                pltpu.async_copy(hl_hbm.at[src_v.at[j]], rows_v, gsem).wait()

                def group_body(g, cc):
                    w16 = w_v[j, pl.ds(g * 16, 16)]
                    for e in range(16):
                        ws = w16[e]
                        r = g * 16 + e
                        for kk in range(D // 16):
                            sl = pl.ds(kk * 16, 16)
                            rows_v[r, sl] = rows_v[r, sl] * ws
                    return cc
                lax.fori_loop(0, C // 16, group_body, 0)

                pltpu.sync_copy(rows_v, acc.at[dst_v.at[j]], add=True)
                return carry2
            lax.fori_loop(0, SB, chunk_body, 0)
            return carry
        lax.fori_loop(0, NSC, super_body, 0)

        plsc.subcore_barrier()
        pltpu.sync_copy(acc.at[pl.ds(s * RPT, RPT)],
                        out_hbm.at[c, pl.ds(s * RPT, RPT)])

        @pl.when(s == 0)
        def _write_rem():
            pltpu.sync_copy(acc.at[pl.ds(NS * RPT, REM)],
                            out_hbm.at[c, pl.ds(NS * RPT, REM)])

    return k(hl, src_r, dst_r, w_r)


_RB = 1000
_GRID = N // _RB


def _tc_first(x, w0t, b0, t0):
    def body(x_ref, w_ref, b_ref, t_ref, hl_ref, hid_ref):
        xv = x_ref[...]
        hid_ref[...] = t_ref[0, 0] * xv
        hl_ref[...] = (jnp.dot(xv, w_ref[...],
                               preferred_element_type=jnp.float32)
                       + b_ref[...])
    return pl.pallas_call(
        body,
        grid=(_GRID,),
        in_specs=[
            pl.BlockSpec((_RB, D), lambda i: (i, 0)),
            pl.BlockSpec((D, D), lambda i: (0, 0)),
            pl.BlockSpec((1, D), lambda i: (0, 0)),
            pl.BlockSpec((1, 1), lambda i: (0, 0)),
        ],
        out_specs=[
            pl.BlockSpec((_RB, D), lambda i: (i, 0)),
            pl.BlockSpec((_RB, D), lambda i: (i, 0)),
        ],
        out_shape=[
            jax.ShapeDtypeStruct((N, D), jnp.float32),
            jax.ShapeDtypeStruct((N, D), jnp.float32),
        ],
    )(x, w0t, b0, t0)


def _tc_mid(p, hidden, wt, bvec, t):
    def body(p_ref, hid_ref, w_ref, b_ref, t_ref, hl_ref, hido_ref):
        h = jnp.maximum(jnp.sum(p_ref[...], axis=0), 0.0)
        hido_ref[...] = hid_ref[...] + t_ref[0, 0] * h
        hl_ref[...] = (jnp.dot(h, w_ref[...],
                               preferred_element_type=jnp.float32)
                       + b_ref[...])
    return pl.pallas_call(
        body,
        grid=(_GRID,),
        in_specs=[
            pl.BlockSpec((NC, _RB, D), lambda i: (0, i, 0)),
            pl.BlockSpec((_RB, D), lambda i: (i, 0)),
            pl.BlockSpec((D, D), lambda i: (0, 0)),
            pl.BlockSpec((1, D), lambda i: (0, 0)),
            pl.BlockSpec((1, 1), lambda i: (0, 0)),
        ],
        out_specs=[
            pl.BlockSpec((_RB, D), lambda i: (i, 0)),
            pl.BlockSpec((_RB, D), lambda i: (i, 0)),
        ],
        out_shape=[
            jax.ShapeDtypeStruct((N, D), jnp.float32),
            jax.ShapeDtypeStruct((N, D), jnp.float32),
        ],
    )(p, hidden, wt, bvec, t)


def _tc_last(p, hidden, t):
    def body(p_ref, hid_ref, t_ref, hido_ref):
        h = jnp.maximum(jnp.sum(p_ref[...], axis=0), 0.0)
        hido_ref[...] = hid_ref[...] + t_ref[0, 0] * h
    return pl.pallas_call(
        body,
        grid=(_GRID,),
        in_specs=[
            pl.BlockSpec((NC, _RB, D), lambda i: (0, i, 0)),
            pl.BlockSpec((_RB, D), lambda i: (i, 0)),
            pl.BlockSpec((1, 1), lambda i: (0, 0)),
        ],
        out_specs=pl.BlockSpec((_RB, D), lambda i: (i, 0)),
        out_shape=jax.ShapeDtypeStruct((N, D), jnp.float32),
    )(p, hidden, t)


def kernel(x, edge_index, edge_weight, W, b, temp):
    src_r = edge_index[0].reshape(NW, NSC, SB, C)
    dst_r = edge_index[1].reshape(NW, NSC, SB, C)
    w_r = edge_weight.reshape(NW, NSC, SB, C)
    wt = jnp.swapaxes(W, 1, 2)
    b2 = b.reshape(L, 1, D)
    tc = temp.reshape(L + 1, 1, 1)

    hl, hidden = _tc_first(x, wt[0], b2[0], tc[0])
    for i in range(1, L):
        p = _sc_aggregate(hl, src_r, dst_r, w_r)
        hl, hidden = _tc_mid(p, hidden, wt[i], b2[i], tc[i])
    p = _sc_aggregate(hl, src_r, dst_r, w_r)
    return _tc_last(p, hidden, tc[L])
